# baseline (speedup 1.0000x reference)
import functools

import jax
import jax.numpy as jnp
from jax import lax
from jax.experimental import pallas as pl
from jax.experimental.pallas import tpu as pltpu
from jax.experimental.pallas import tpu_sc as plsc

_CH = 128
_NBUF = 2


@functools.cache
def _build(n_chunks_total, d):
    info = plsc.get_sparse_core_info()
    nc, ns = info.num_cores, info.num_subcores
    nw = nc * ns
    nch = n_chunks_total // nw
    assert nch * nw == n_chunks_total
    b_tot = n_chunks_total * _CH
    mesh = plsc.VectorSubcoreMesh(core_axis_name="c", subcore_axis_name="s")

    def body(idx_hbm, table_hbm, out_hbm, idx_v, rows_v, gsem):
        wid = lax.axis_index("s") * nc + lax.axis_index("c")
        c0 = wid * nch
        pltpu.sync_copy(idx_hbm.at[wid], idx_v)

        def gather(jj, b):
            return pltpu.make_async_copy(
                table_hbm.at[idx_v.at[jj]], rows_v.at[b], gsem)

        for b in range(_NBUF):
            gather(b, b).start()

        @pl.loop(0, nch, step=_NBUF)
        def _(j):
            for b in range(_NBUF):
                jj = j + b
                gather(jj, b).wait()
                pltpu.sync_copy(rows_v.at[b],
                                out_hbm.at[pl.ds((c0 + jj) * _CH, _CH)])
                nxt = jj + _NBUF

                @pl.when(nxt < nch)
                def _():
                    gather(nxt, b).start()

    return pl.kernel(
        body,
        out_type=jax.ShapeDtypeStruct((b_tot, d), jnp.float32),
        mesh=mesh,
        compiler_params=pltpu.CompilerParams(use_tc_tiling_on_sc=False),
        scratch_types=[
            pltpu.VMEM((nch, _CH), jnp.int32),
            pltpu.VMEM((_NBUF, _CH, d), jnp.float32),
            pltpu.SemaphoreType.DMA,
        ],
    )


def kernel(indices, table):
    b, l = indices.shape
    _, d = table.shape
    flat = indices.reshape(-1).astype(jnp.int32)
    n_chunks_total = flat.shape[0] // _CH
    info = plsc.get_sparse_core_info()
    nw = info.num_cores * info.num_subcores
    idx3d = flat.reshape(nw, n_chunks_total // nw, _CH)
    out = _build(n_chunks_total, d)(idx3d, table)
    return out.reshape(b, l, d)


# speedup vs baseline: 1.0110x; 1.0110x over previous
import functools

import jax
import jax.numpy as jnp
from jax import lax
from jax.experimental import pallas as pl
from jax.experimental.pallas import tpu as pltpu
from jax.experimental.pallas import tpu_sc as plsc

_CH = 640
_NBUF = 2


@functools.cache
def _build(n_chunks_total, d):
    info = plsc.get_sparse_core_info()
    nc, ns = info.num_cores, info.num_subcores
    nw = nc * ns
    nch = n_chunks_total // nw
    assert nch * nw == n_chunks_total
    b_tot = n_chunks_total * _CH
    mesh = plsc.VectorSubcoreMesh(core_axis_name="c", subcore_axis_name="s")

    def body(idx_hbm, table_hbm, out_hbm, idx_v, rows_v, gsem):
        wid = lax.axis_index("s") * nc + lax.axis_index("c")
        c0 = wid * nch
        pltpu.sync_copy(idx_hbm.at[wid], idx_v)

        def gather(jj, b):
            return pltpu.make_async_copy(
                table_hbm.at[idx_v.at[jj]], rows_v.at[b], gsem)

        for b in range(_NBUF):
            gather(b, b).start()

        @pl.loop(0, nch, step=_NBUF)
        def _(j):
            for b in range(_NBUF):
                jj = j + b
                gather(jj, b).wait()
                pltpu.sync_copy(rows_v.at[b],
                                out_hbm.at[pl.ds((c0 + jj) * _CH, _CH)])
                nxt = jj + _NBUF

                @pl.when(nxt < nch)
                def _():
                    gather(nxt, b).start()

    return pl.kernel(
        body,
        out_type=jax.ShapeDtypeStruct((b_tot, d), jnp.float32),
        mesh=mesh,
        compiler_params=pltpu.CompilerParams(use_tc_tiling_on_sc=False),
        scratch_types=[
            pltpu.VMEM((nch, _CH), jnp.int32),
            pltpu.VMEM((_NBUF, _CH, d), jnp.float32),
            pltpu.SemaphoreType.DMA,
        ],
    )


def kernel(indices, table):
    b, l = indices.shape
    _, d = table.shape
    flat = indices.reshape(-1).astype(jnp.int32)
    n_chunks_total = flat.shape[0] // _CH
    info = plsc.get_sparse_core_info()
    nw = info.num_cores * info.num_subcores
    idx3d = flat.reshape(nw, n_chunks_total // nw, _CH)
    out = _build(n_chunks_total, d)(idx3d, table)
    return out.reshape(b, l, d)


# speedup vs baseline: 1.0259x; 1.0147x over previous
import functools

import jax
import jax.numpy as jnp
from jax import lax
from jax.experimental import pallas as pl
from jax.experimental.pallas import tpu as pltpu
from jax.experimental.pallas import tpu_sc as plsc

_CH = 640
_NBUF = 2


@functools.cache
def _build(n_chunks_total, d):
    info = plsc.get_sparse_core_info()
    nc, ns = info.num_cores, info.num_subcores
    nw = nc * ns
    nch = n_chunks_total // nw
    assert nch * nw == n_chunks_total
    b_tot = n_chunks_total * _CH
    mesh = plsc.VectorSubcoreMesh(core_axis_name="c", subcore_axis_name="s")

    def body(idx_hbm, table_hbm, out_hbm, idx_v, rows_v, gsem):
        wid = lax.axis_index("s") * nc + lax.axis_index("c")
        c0 = wid * nch
        pltpu.sync_copy(idx_hbm.at[wid], idx_v)

        def gather(jj, b):
            return pltpu.make_async_copy(
                table_hbm.at[idx_v.at[jj]], rows_v.at[b], gsem)

        for b in range(_NBUF):
            gather(b, b).start()

        @pl.loop(0, nch, step=_NBUF)
        def _(j):
            for b in range(_NBUF):
                jj = j + b
                gather(jj, b).wait()
                pltpu.sync_copy(rows_v.at[b],
                                out_hbm.at[pl.ds((c0 + jj) * _CH, _CH)])
                nxt = jj + _NBUF

                @pl.when(nxt < nch)
                def _():
                    gather(nxt, b).start()

    return pl.kernel(
        body,
        out_type=jax.ShapeDtypeStruct((b_tot, d), jnp.float32),
        mesh=mesh,
        compiler_params=pltpu.CompilerParams(use_tc_tiling_on_sc=False),
        scratch_types=[
            pltpu.VMEM((nch, _CH), jnp.int32),
            pltpu.VMEM((_NBUF, _CH, d), jnp.float32),
            pltpu.SemaphoreType.DMA,
        ],
    )


def kernel(indices, table):
    b, l = indices.shape
    _, d = table.shape
    flat = indices.T.reshape(-1).astype(jnp.int32)
    n_chunks_total = flat.shape[0] // _CH
    info = plsc.get_sparse_core_info()
    nw = info.num_cores * info.num_subcores
    idx3d = flat.reshape(nw, n_chunks_total // nw, _CH)
    out = _build(n_chunks_total, d)(idx3d, table)
    return out.reshape(l, b, d).transpose(1, 0, 2)


# speedup vs baseline: 1.1890x; 1.1590x over previous
import functools

import jax
import jax.numpy as jnp
from jax import lax
from jax.experimental import pallas as pl
from jax.experimental.pallas import tpu as pltpu
from jax.experimental.pallas import tpu_sc as plsc

_CH = 320
_NBUF = 2
_DP = 128


@functools.cache
def _build(n_tok, v):
    info = plsc.get_sparse_core_info()
    nc, ns = info.num_cores, info.num_subcores
    nw = nc * ns
    nch = n_tok // (nw * _CH)
    assert nch * nw * _CH == n_tok
    assert nch % _NBUF == 0
    mesh = plsc.VectorSubcoreMesh(core_axis_name="c", subcore_axis_name="s")

    def body(idx_hbm, tab_hbm, out_hbm, idx_v, rows_v, gsem):
        wid = lax.axis_index("s") * nc + lax.axis_index("c")
        c0 = wid * nch
        pltpu.sync_copy(idx_hbm.at[wid], idx_v)

        def gather(jj, b):
            return pltpu.make_async_copy(
                tab_hbm.at[idx_v.at[jj]], rows_v.at[b], gsem)

        for b in range(_NBUF):
            gather(b, b).start()

        @pl.loop(0, nch, step=_NBUF)
        def _(j):
            for b in range(_NBUF):
                jj = j + b
                gather(jj, b).wait()
                pltpu.sync_copy(rows_v.at[b],
                                out_hbm.at[pl.ds((c0 + jj) * _CH, _CH)])
                nxt = jj + _NBUF

                @pl.when(nxt < nch)
                def _():
                    gather(nxt, b).start()

    return pl.kernel(
        body,
        out_type=jax.ShapeDtypeStruct((n_tok, _DP), jnp.float32),
        mesh=mesh,
        compiler_params=pltpu.CompilerParams(use_tc_tiling_on_sc=False),
        scratch_types=[
            pltpu.VMEM((nch, _CH), jnp.int32),
            pltpu.VMEM((_NBUF, _CH, _DP), jnp.float32),
            pltpu.SemaphoreType.DMA,
        ],
    )


def kernel(indices, table):
    b, l = indices.shape
    v, d = table.shape
    flat = indices.T.reshape(-1).astype(jnp.int32)
    n_tok = flat.shape[0]
    info = plsc.get_sparse_core_info()
    nw = info.num_cores * info.num_subcores
    idx3d = flat.reshape(nw, n_tok // (nw * _CH), _CH)
    tab_pad = jnp.pad(table, ((0, 0), (0, _DP - d)))
    out = _build(n_tok, v)(idx3d, tab_pad)
    return out[:, :d].reshape(l, b, d).transpose(1, 0, 2)
